# Initial kernel scaffold; baseline (speedup 1.0000x reference)
import jax
import jax.numpy as jnp
from jax.experimental import pallas as pl


def kernel(xyz):
    raise NotImplementedError("write your pallas kernel here")


# speedup vs baseline: 1.1751x; 1.1751x over previous
import jax
import jax.numpy as jnp
from jax import lax
from jax.experimental import pallas as pl
from jax.experimental.pallas import tpu as pltpu

_B, _N = 4, 4096
_TILE = 512


def _symloss_body(q_ref, t_ref, out_ref):
    b = pl.program_id(0)
    i = pl.program_id(1)
    q = q_ref[0]
    qx = q[:, 0:1]
    qy = q[:, 1:2]
    qz = q[:, 2:3]
    qm = jnp.concatenate([-qx, qy, qz], axis=1)
    t = t_ref[0]
    tx = t[0:1, :]
    ty = t[1:2, :]
    tz = t[2:3, :]
    qn = (qx * qx + qy * qy) + qz * qz
    tn = (tx * tx + ty * ty) + tz * tz
    ab = lax.dot_general(qm, t, (((1,), (0,)), ((), ())),
                         preferred_element_type=jnp.float32)
    d2 = (qn + tn) - 2.0 * ab
    s = jnp.sum(jnp.min(d2, axis=1))

    @pl.when((b == 0) & (i == 0))
    def _init():
        out_ref[0, 0] = 0.0

    out_ref[0, 0] += s


def kernel(xyz):
    B, N, _ = xyz.shape
    t = jnp.transpose(xyz, (0, 2, 1))
    total = pl.pallas_call(
        _symloss_body,
        grid=(B, N // _TILE),
        in_specs=[
            pl.BlockSpec((1, _TILE, 3), lambda b, i: (b, i, 0)),
            pl.BlockSpec((1, 3, N), lambda b, i: (b, 0, 0)),
        ],
        out_specs=pl.BlockSpec((1, 1), lambda b, i: (0, 0),
                               memory_space=pltpu.SMEM),
        out_shape=jax.ShapeDtypeStruct((1, 1), jnp.float32),
    )(xyz, t)
    return total[0, 0] * (2.0 / (B * N))


# speedup vs baseline: 1.2587x; 1.0712x over previous
import jax
import jax.numpy as jnp
from jax import lax
from jax.experimental import pallas as pl
from jax.experimental.pallas import tpu as pltpu

_B, _N = 4, 4096
_TILE = 512


def _symloss_body(q_ref, t_ref, out_ref):
    b = pl.program_id(0)
    i = pl.program_id(1)
    q = q_ref[0]
    qx = q[:, 0:1]
    qy = q[:, 1:2]
    qz = q[:, 2:3]
    t = t_ref[0]
    tx = t[0:1, :]
    ty = t[1:2, :]
    tz = t[2:3, :]
    qn = (qx * qx + qy * qy) + qz * qz
    tn = (tx * tx + ty * ty) + tz * tz
    a2 = jnp.concatenate([2.0 * qx, -2.0 * qy, -2.0 * qz], axis=1)
    ab2 = lax.dot_general(a2, t, (((1,), (0,)), ((), ())),
                          preferred_element_type=jnp.float32)
    e = tn + ab2
    s = jnp.sum(qn) + jnp.sum(jnp.min(e, axis=1))

    @pl.when((b == 0) & (i == 0))
    def _init():
        out_ref[0, 0] = 0.0

    out_ref[0, 0] += s


def kernel(xyz):
    B, N, _ = xyz.shape
    t = jnp.transpose(xyz, (0, 2, 1))
    total = pl.pallas_call(
        _symloss_body,
        grid=(B, N // _TILE),
        in_specs=[
            pl.BlockSpec((1, _TILE, 3), lambda b, i: (b, i, 0)),
            pl.BlockSpec((1, 3, N), lambda b, i: (b, 0, 0)),
        ],
        out_specs=pl.BlockSpec((1, 1), lambda b, i: (0, 0),
                               memory_space=pltpu.SMEM),
        out_shape=jax.ShapeDtypeStruct((1, 1), jnp.float32),
    )(xyz, t)
    return total[0, 0] * (2.0 / (B * N))


# speedup vs baseline: 1.2783x; 1.0156x over previous
import jax
import jax.numpy as jnp
from jax import lax
from jax.experimental import pallas as pl
from jax.experimental.pallas import tpu as pltpu

_B, _N = 4, 4096
_TILE = 512


def _symloss_body(q_ref, t_ref, out_ref):
    b = pl.program_id(0)
    i = pl.program_id(1)
    q = q_ref[0]
    qx = q[:, 0:1]
    qy = q[:, 1:2]
    qz = q[:, 2:3]
    t = t_ref[0]
    tx = t[0:1, :]
    ty = t[1:2, :]
    tz = t[2:3, :]
    qn = (qx * qx + qy * qy) + qz * qz
    tn = (tx * tx + ty * ty) + tz * tz
    a2 = jnp.concatenate([2.0 * qx, -2.0 * qy, -2.0 * qz], axis=1)
    ab2 = lax.dot_general(a2.astype(jnp.bfloat16), t.astype(jnp.bfloat16),
                          (((1,), (0,)), ((), ())),
                          preferred_element_type=jnp.float32)
    e = tn + ab2
    s = jnp.sum(qn) + jnp.sum(jnp.min(e, axis=1))

    @pl.when((b == 0) & (i == 0))
    def _init():
        out_ref[0, 0] = 0.0

    out_ref[0, 0] += s


def kernel(xyz):
    B, N, _ = xyz.shape
    t = jnp.transpose(xyz, (0, 2, 1))
    total = pl.pallas_call(
        _symloss_body,
        grid=(B, N // _TILE),
        in_specs=[
            pl.BlockSpec((1, _TILE, 3), lambda b, i: (b, i, 0)),
            pl.BlockSpec((1, 3, N), lambda b, i: (b, 0, 0)),
        ],
        out_specs=pl.BlockSpec((1, 1), lambda b, i: (0, 0),
                               memory_space=pltpu.SMEM),
        out_shape=jax.ShapeDtypeStruct((1, 1), jnp.float32),
    )(xyz, t)
    return total[0, 0] * (2.0 / (B * N))


# speedup vs baseline: 1.4088x; 1.1020x over previous
import jax
import jax.numpy as jnp
from jax import lax
from jax.experimental import pallas as pl
from jax.experimental.pallas import tpu as pltpu

_B, _N = 4, 4096
_TILE = 1024


def _symloss_body(q_ref, t_ref, out_ref):
    b = pl.program_id(0)
    i = pl.program_id(1)
    q = q_ref[0]
    qx = q[:, 0:1]
    qy = q[:, 1:2]
    qz = q[:, 2:3]
    t = t_ref[0]
    tx = t[0:1, :]
    ty = t[1:2, :]
    tz = t[2:3, :]
    qn = (qx * qx + qy * qy) + qz * qz
    tn = (tx * tx + ty * ty) + tz * tz
    a2 = jnp.concatenate([2.0 * qx, -2.0 * qy, -2.0 * qz], axis=1)
    ab2 = lax.dot_general(a2.astype(jnp.bfloat16), t.astype(jnp.bfloat16),
                          (((1,), (0,)), ((), ())),
                          preferred_element_type=jnp.float32)
    e = tn + ab2
    s = jnp.sum(qn) + jnp.sum(jnp.min(e, axis=1))

    @pl.when((b == 0) & (i == 0))
    def _init():
        out_ref[0, 0] = 0.0

    out_ref[0, 0] += s


def kernel(xyz):
    B, N, _ = xyz.shape
    t = jnp.transpose(xyz, (0, 2, 1))
    total = pl.pallas_call(
        _symloss_body,
        grid=(B, N // _TILE),
        in_specs=[
            pl.BlockSpec((1, _TILE, 3), lambda b, i: (b, i, 0)),
            pl.BlockSpec((1, 3, N), lambda b, i: (b, 0, 0)),
        ],
        out_specs=pl.BlockSpec((1, 1), lambda b, i: (0, 0),
                               memory_space=pltpu.SMEM),
        out_shape=jax.ShapeDtypeStruct((1, 1), jnp.float32),
    )(xyz, t)
    return total[0, 0] * (2.0 / (B * N))


# speedup vs baseline: 1.4789x; 1.0498x over previous
import jax
import jax.numpy as jnp
from jax import lax
from jax.experimental import pallas as pl
from jax.experimental.pallas import tpu as pltpu

_B, _N = 4, 4096
_TILE = 2048


def _symloss_body(q_ref, t_ref, out_ref):
    b = pl.program_id(0)
    i = pl.program_id(1)
    q = q_ref[0]
    qx = q[:, 0:1]
    qy = q[:, 1:2]
    qz = q[:, 2:3]
    t = t_ref[0]
    tx = t[0:1, :]
    ty = t[1:2, :]
    tz = t[2:3, :]
    qn = (qx * qx + qy * qy) + qz * qz
    tn = (tx * tx + ty * ty) + tz * tz
    a2 = jnp.concatenate([2.0 * qx, -2.0 * qy, -2.0 * qz], axis=1)
    ab2 = lax.dot_general(a2.astype(jnp.bfloat16), t.astype(jnp.bfloat16),
                          (((1,), (0,)), ((), ())),
                          preferred_element_type=jnp.float32)
    e = tn + ab2
    s = jnp.sum(qn) + jnp.sum(jnp.min(e, axis=1))

    @pl.when((b == 0) & (i == 0))
    def _init():
        out_ref[0, 0] = 0.0

    out_ref[0, 0] += s


def kernel(xyz):
    B, N, _ = xyz.shape
    t = jnp.transpose(xyz, (0, 2, 1))
    total = pl.pallas_call(
        _symloss_body,
        grid=(B, N // _TILE),
        in_specs=[
            pl.BlockSpec((1, _TILE, 3), lambda b, i: (b, i, 0)),
            pl.BlockSpec((1, 3, N), lambda b, i: (b, 0, 0)),
        ],
        out_specs=pl.BlockSpec((1, 1), lambda b, i: (0, 0),
                               memory_space=pltpu.SMEM),
        out_shape=jax.ShapeDtypeStruct((1, 1), jnp.float32),
    )(xyz, t)
    return total[0, 0] * (2.0 / (B * N))


# speedup vs baseline: 1.5036x; 1.0167x over previous
import jax
import jax.numpy as jnp
from jax import lax
from jax.experimental import pallas as pl
from jax.experimental.pallas import tpu as pltpu

_B, _N = 4, 4096
_TILE = 2048


def _symloss_body(q_ref, t_ref, out_ref):
    b = pl.program_id(0)
    i = pl.program_id(1)
    q = q_ref[0]
    qx = q[:, 0:1]
    qy = q[:, 1:2]
    qz = q[:, 2:3]
    t = t_ref[0]
    qn = (qx * qx + qy * qy) + qz * qz
    tn = jnp.sum(t * t, axis=1, keepdims=True)
    tn_hi = tn.astype(jnp.bfloat16).astype(jnp.float32)
    tn_lo = tn - tn_hi
    ones = jnp.ones_like(qx)
    a_aug = jnp.concatenate(
        [2.0 * qx, -2.0 * qy, -2.0 * qz, ones, ones], axis=1)
    t_aug = jnp.concatenate([t, tn_hi, tn_lo], axis=1)
    e = lax.dot_general(a_aug.astype(jnp.bfloat16),
                        t_aug.astype(jnp.bfloat16),
                        (((1,), (1,)), ((), ())),
                        preferred_element_type=jnp.float32)
    s = jnp.sum(qn) + jnp.sum(jnp.min(e, axis=1))

    @pl.when((b == 0) & (i == 0))
    def _init():
        out_ref[0, 0] = 0.0

    out_ref[0, 0] += s


def kernel(xyz):
    B, N, _ = xyz.shape
    total = pl.pallas_call(
        _symloss_body,
        grid=(B, N // _TILE),
        in_specs=[
            pl.BlockSpec((1, _TILE, 3), lambda b, i: (b, i, 0)),
            pl.BlockSpec((1, N, 3), lambda b, i: (b, 0, 0)),
        ],
        out_specs=pl.BlockSpec((1, 1), lambda b, i: (0, 0),
                               memory_space=pltpu.SMEM),
        out_shape=jax.ShapeDtypeStruct((1, 1), jnp.float32),
    )(xyz, xyz)
    return total[0, 0] * (2.0 / (B * N))
